# baseline (speedup 1.0000x reference)
import functools

import jax
import jax.numpy as jnp
from jax import lax
from jax.experimental import pallas as pl
from jax.experimental.pallas import tpu as pltpu
from jax.experimental.pallas import tpu_sc as plsc

NC = 2
NS = 16

N = 10000
NP = 10240
E = 320000
EP = 327680
CHN = 1024
CHUNKS = EP // CHN
RPS = NP // NS
DW = 8

SLOW_CID = 1
AGG_SLOW = 6
AGG_FAST = (CHUNKS - NS * AGG_SLOW) // NS
DEG_SLOW = 8
DEG_FAST = (CHUNKS - NS * DEG_SLOW) // NS


def _sc_mesh():
  return plsc.VectorSubcoreMesh(core_axis_name="c", subcore_axis_name="s")


def _zero_acc(acc_sh, zeros_hbm, sid):
  pltpu.sync_copy(zeros_hbm.at[pl.ds(sid * RPS, RPS)],
                  acc_sh.at[pl.ds(sid * RPS, RPS)])


def _copy_out(acc_sh, out_hbm, cid, sid):
  pltpu.sync_copy(acc_sh.at[pl.ds(sid * RPS, RPS)],
                  out_hbm.at[cid, pl.ds(sid * RPS, RPS)])


def _tile_span(cid, sid, n_slow, n_fast):
  slow = cid == SLOW_CID
  start = jnp.where(slow, sid * n_slow, NS * n_slow + sid * n_fast)
  count = jnp.where(slow, n_slow, n_fast)
  return start, count


def _deg_body(dst_hbm, ones_hbm, zeros_hbm, out_hbm, dstv, ones_v, acc_sh,
              sem):
  cid = lax.axis_index("c")
  sid = lax.axis_index("s")
  _zero_acc(acc_sh, zeros_hbm, sid)
  start, count = _tile_span(cid, sid, DEG_SLOW, DEG_FAST)
  pltpu.sync_copy(dst_hbm.at[pl.ds(start, DEG_FAST)], dstv)
  pltpu.sync_copy(ones_hbm, ones_v)
  plsc.subcore_barrier()

  def fire(c, _):
    pltpu.async_copy(ones_v, acc_sh.at[dstv.at[c]], sem, add=True)
    return 0

  def drain(c, _):
    pltpu.make_async_copy(ones_v, acc_sh.at[dstv.at[0]], sem).wait()
    return 0

  lax.fori_loop(0, count, fire, 0)
  lax.fori_loop(0, count, drain, 0)
  plsc.subcore_barrier()
  _copy_out(acc_sh, out_hbm, cid, sid)


def _make_deg_kernel():
  return pl.kernel(
      _deg_body,
      out_type=jax.ShapeDtypeStruct((NC, NP, DW), jnp.float32),
      mesh=_sc_mesh(),
      scratch_types=[
          pltpu.VMEM((DEG_FAST, CHN), jnp.int32),
          pltpu.VMEM((CHN, DW), jnp.float32),
          pltpu.VMEM_SHARED((NP, DW), jnp.float32),
          pltpu.SemaphoreType.DMA,
      ],
      compiler_params=pltpu.CompilerParams(use_tc_tiling_on_sc=False),
  )


def _agg_body(w, src_hbm, dst_hbm, g_hbm, zeros_hbm, out_hbm, srcv, dstv,
              rows, acc_sh, sg0, sg1, ss0, ss1):
  cid = lax.axis_index("c")
  sid = lax.axis_index("s")
  _zero_acc(acc_sh, zeros_hbm, sid)
  start, count = _tile_span(cid, sid, AGG_SLOW, AGG_FAST)
  pltpu.sync_copy(src_hbm.at[pl.ds(start, AGG_FAST)], srcv)
  pltpu.sync_copy(dst_hbm.at[pl.ds(start, AGG_FAST)], dstv)
  plsc.subcore_barrier()

  def body(i, _):
    k0 = 2 * i
    k1 = k0 + 1

    @pl.when(i > 0)
    def _():
      pltpu.make_async_copy(rows.at[0], acc_sh.at[dstv.at[k0]], ss0).wait()

    pltpu.async_copy(g_hbm.at[srcv.at[k0]], rows.at[0], sg0)

    @pl.when(i > 0)
    def _():
      pltpu.make_async_copy(rows.at[1], acc_sh.at[dstv.at[k1]], ss1).wait()

    pltpu.async_copy(g_hbm.at[srcv.at[k1]], rows.at[1], sg1)

    pltpu.make_async_copy(g_hbm.at[srcv.at[k0]], rows.at[0], sg0).wait()
    pltpu.async_copy(rows.at[0], acc_sh.at[dstv.at[k0]], ss0, add=True)
    pltpu.make_async_copy(g_hbm.at[srcv.at[k1]], rows.at[1], sg1).wait()
    pltpu.async_copy(rows.at[1], acc_sh.at[dstv.at[k1]], ss1, add=True)
    return 0

  lax.fori_loop(0, count // 2, body, 0)
  pltpu.make_async_copy(rows.at[0], acc_sh.at[dstv.at[0]], ss0).wait()
  pltpu.make_async_copy(rows.at[1], acc_sh.at[dstv.at[1]], ss1).wait()
  plsc.subcore_barrier()
  _copy_out(acc_sh, out_hbm, cid, sid)


def _make_agg_kernel(w):
  return pl.kernel(
      functools.partial(_agg_body, w),
      out_type=jax.ShapeDtypeStruct((NC, NP, w), jnp.float32),
      mesh=_sc_mesh(),
      scratch_types=[
          pltpu.VMEM((AGG_FAST, CHN), jnp.int32),
          pltpu.VMEM((AGG_FAST, CHN), jnp.int32),
          pltpu.VMEM((2, CHN, w), jnp.float32),
          pltpu.VMEM_SHARED((NP, w), jnp.float32),
          pltpu.SemaphoreType.DMA,
          pltpu.SemaphoreType.DMA,
          pltpu.SemaphoreType.DMA,
          pltpu.SemaphoreType.DMA,
      ],
      compiler_params=pltpu.CompilerParams(use_tc_tiling_on_sc=False),
  )


def _mm_body(x_ref, w1_ref, h_ref):
  h_ref[...] = jnp.dot(x_ref[...], w1_ref[...],
                       preferred_element_type=jnp.float32)


def _prep_body(h_ref, degp_ref, g1_ref, d_ref):
  deg = degp_ref[0, :N, 0:1] + degp_ref[1, :N, 0:1] + 1.0
  d = lax.rsqrt(deg)
  g1_ref[...] = h_ref[...] * d
  d_ref[...] = d


def _mid_body(acc_ref, g1_ref, d_ref, b1_ref, w2_ref, g2_ref):
  d = d_ref[...]
  acc = acc_ref[0, :N, :] + acc_ref[1, :N, :] + g1_ref[...]
  out1 = jnp.maximum(acc * d + b1_ref[...], 0.0)
  h2 = jnp.dot(out1, w2_ref[...], preferred_element_type=jnp.float32)
  g2_ref[...] = h2 * d


def _fin_body(acc_ref, g2_ref, d_ref, b2_ref, out_ref):
  d = d_ref[...]
  out2 = (acc_ref[0, :N, :] + acc_ref[1, :N, :] + g2_ref[...]) * d
  out2 = out2 + b2_ref[...]
  m = jnp.sum(out2, axis=0, keepdims=True) * (1.0 / N)
  mask = lax.broadcasted_iota(jnp.int32, (1, 8), 1) < 6
  mx = jnp.max(jnp.where(mask, m, -1e30), axis=1, keepdims=True)
  z = m - mx
  s = jnp.sum(jnp.where(mask, jnp.exp(z), 0.0), axis=1, keepdims=True)
  out_ref[...] = z - jnp.log(s)


def kernel(x, edge_index, W1, b1, W2, b2):
  src = edge_index[0]
  dst = edge_index[1]
  pad = EP - E
  srcp = jnp.concatenate([src, jnp.zeros((pad,), jnp.int32)])
  dstp = jnp.concatenate([dst, jnp.full((pad,), N, jnp.int32)])
  src2d = srcp.reshape(CHUNKS, CHN)
  dst2d = dstp.reshape(CHUNKS, CHN)

  w2p = jnp.zeros((16, 8), jnp.float32).at[:, :6].set(W2)
  b2p = jnp.zeros((1, 8), jnp.float32).at[0, :6].set(b2)
  b1r = b1.reshape(1, 16)

  ones_deg = jnp.ones((CHN, DW), jnp.float32)
  zeros_deg = jnp.zeros((NP, DW), jnp.float32)
  zeros16 = jnp.zeros((NP, 16), jnp.float32)
  zeros8 = jnp.zeros((NP, 8), jnp.float32)

  h1 = pl.pallas_call(
      _mm_body,
      out_shape=jax.ShapeDtypeStruct((N, 16), jnp.float32),
  )(x, W1)

  deg_parts = _make_deg_kernel()(dst2d, ones_deg, zeros_deg)

  g1, d = pl.pallas_call(
      _prep_body,
      out_shape=[
          jax.ShapeDtypeStruct((N, 16), jnp.float32),
          jax.ShapeDtypeStruct((N, 1), jnp.float32),
      ],
  )(h1, deg_parts)

  acc1 = _make_agg_kernel(16)(src2d, dst2d, g1, zeros16)

  g2 = pl.pallas_call(
      _mid_body,
      out_shape=jax.ShapeDtypeStruct((N, 8), jnp.float32),
  )(acc1, g1, d, b1r, w2p)

  acc2 = _make_agg_kernel(8)(src2d, dst2d, g2, zeros8)

  outp = pl.pallas_call(
      _fin_body,
      out_shape=jax.ShapeDtypeStruct((1, 8), jnp.float32),
  )(acc2, g2, d, b2p)

  return outp[:, :6]


# speedup vs baseline: 1.0743x; 1.0743x over previous
import functools

import jax
import jax.numpy as jnp
from jax import lax
from jax.experimental import pallas as pl
from jax.experimental.pallas import tpu as pltpu
from jax.experimental.pallas import tpu_sc as plsc

NC = 2
NS = 16

N = 10000
NP = 10240
E = 320000
EP = 327680
CHN = 1024
CHUNKS = EP // CHN
RPS = NP // NS
DW = 8

SLOW_CID = 1
AGG_SLOW = 10
AGG_FAST = (CHUNKS - NS * AGG_SLOW) // NS
DEG_SLOW = 10
DEG_FAST = (CHUNKS - NS * DEG_SLOW) // NS


def _sc_mesh():
  return plsc.VectorSubcoreMesh(core_axis_name="c", subcore_axis_name="s")


def _zero_acc(acc_sh, zeros_hbm, sid):
  pltpu.sync_copy(zeros_hbm.at[pl.ds(sid * RPS, RPS)],
                  acc_sh.at[pl.ds(sid * RPS, RPS)])


def _copy_out(acc_sh, out_hbm, cid, sid):
  pltpu.sync_copy(acc_sh.at[pl.ds(sid * RPS, RPS)],
                  out_hbm.at[cid, pl.ds(sid * RPS, RPS)])


def _tile_span(cid, sid, n_slow, n_fast):
  slow = cid == SLOW_CID
  start = jnp.where(slow, sid * n_slow, NS * n_slow + sid * n_fast)
  count = jnp.where(slow, n_slow, n_fast)
  return start, count


def _deg_body(dst_hbm, ones_hbm, zeros_hbm, out_hbm, dstv, ones_v, acc_sh,
              sem):
  cid = lax.axis_index("c")
  sid = lax.axis_index("s")
  _zero_acc(acc_sh, zeros_hbm, sid)
  start, count = _tile_span(cid, sid, DEG_SLOW, DEG_FAST)
  pltpu.sync_copy(dst_hbm.at[pl.ds(start, DEG_FAST)], dstv)
  pltpu.sync_copy(ones_hbm, ones_v)
  plsc.subcore_barrier()

  def fire(c, _):
    pltpu.async_copy(ones_v, acc_sh.at[dstv.at[c]], sem, add=True)
    return 0

  def drain(c, _):
    pltpu.make_async_copy(ones_v, acc_sh.at[dstv.at[0]], sem).wait()
    return 0

  lax.fori_loop(0, count, fire, 0)
  lax.fori_loop(0, count, drain, 0)
  plsc.subcore_barrier()
  _copy_out(acc_sh, out_hbm, cid, sid)


def _make_deg_kernel():
  return pl.kernel(
      _deg_body,
      out_type=jax.ShapeDtypeStruct((NC, NP, DW), jnp.float32),
      mesh=_sc_mesh(),
      scratch_types=[
          pltpu.VMEM((DEG_FAST, CHN), jnp.int32),
          pltpu.VMEM((CHN, DW), jnp.float32),
          pltpu.VMEM_SHARED((NP, DW), jnp.float32),
          pltpu.SemaphoreType.DMA,
      ],
      compiler_params=pltpu.CompilerParams(use_tc_tiling_on_sc=False),
  )


def _agg_body(w, src_hbm, dst_hbm, g_hbm, zeros_hbm, out_hbm, srcv, dstv,
              rows, acc_sh, sg0, sg1, ss0, ss1):
  cid = lax.axis_index("c")
  sid = lax.axis_index("s")
  _zero_acc(acc_sh, zeros_hbm, sid)
  start, count = _tile_span(cid, sid, AGG_SLOW, AGG_FAST)
  pltpu.sync_copy(src_hbm.at[pl.ds(start, AGG_FAST)], srcv)
  pltpu.sync_copy(dst_hbm.at[pl.ds(start, AGG_FAST)], dstv)
  plsc.subcore_barrier()

  def body(i, _):
    k0 = 2 * i
    k1 = k0 + 1

    @pl.when(i > 0)
    def _():
      pltpu.make_async_copy(rows.at[0], acc_sh.at[dstv.at[k0]], ss0).wait()

    pltpu.async_copy(g_hbm.at[srcv.at[k0]], rows.at[0], sg0)

    @pl.when(i > 0)
    def _():
      pltpu.make_async_copy(rows.at[1], acc_sh.at[dstv.at[k1]], ss1).wait()

    pltpu.async_copy(g_hbm.at[srcv.at[k1]], rows.at[1], sg1)

    pltpu.make_async_copy(g_hbm.at[srcv.at[k0]], rows.at[0], sg0).wait()
    pltpu.async_copy(rows.at[0], acc_sh.at[dstv.at[k0]], ss0, add=True)
    pltpu.make_async_copy(g_hbm.at[srcv.at[k1]], rows.at[1], sg1).wait()
    pltpu.async_copy(rows.at[1], acc_sh.at[dstv.at[k1]], ss1, add=True)
    return 0

  lax.fori_loop(0, count // 2, body, 0)
  pltpu.make_async_copy(rows.at[0], acc_sh.at[dstv.at[0]], ss0).wait()
  pltpu.make_async_copy(rows.at[1], acc_sh.at[dstv.at[1]], ss1).wait()
  plsc.subcore_barrier()
  _copy_out(acc_sh, out_hbm, cid, sid)


def _make_agg_kernel(w):
  return pl.kernel(
      functools.partial(_agg_body, w),
      out_type=jax.ShapeDtypeStruct((NC, NP, w), jnp.float32),
      mesh=_sc_mesh(),
      scratch_types=[
          pltpu.VMEM((AGG_FAST, CHN), jnp.int32),
          pltpu.VMEM((AGG_FAST, CHN), jnp.int32),
          pltpu.VMEM((2, CHN, w), jnp.float32),
          pltpu.VMEM_SHARED((NP, w), jnp.float32),
          pltpu.SemaphoreType.DMA,
          pltpu.SemaphoreType.DMA,
          pltpu.SemaphoreType.DMA,
          pltpu.SemaphoreType.DMA,
      ],
      compiler_params=pltpu.CompilerParams(use_tc_tiling_on_sc=False),
  )


def _prep_body(x_ref, w1_ref, degp_ref, g1_ref, d_ref):
  deg = degp_ref[0, :N, 0:1] + degp_ref[1, :N, 0:1] + 1.0
  d = lax.rsqrt(deg)
  h = jnp.dot(x_ref[...], w1_ref[...], preferred_element_type=jnp.float32)
  g1_ref[...] = h * d
  d_ref[...] = d


def _mid_body(acc_ref, g1_ref, d_ref, b1_ref, w2_ref, g2_ref):
  d = d_ref[...]
  acc = acc_ref[0, :N, :] + acc_ref[1, :N, :] + g1_ref[...]
  out1 = jnp.maximum(acc * d + b1_ref[...], 0.0)
  h2 = jnp.dot(out1, w2_ref[...], preferred_element_type=jnp.float32)
  g2_ref[...] = h2 * d


def _fin_body(acc_ref, g2_ref, d_ref, b2_ref, out_ref):
  d = d_ref[...]
  out2 = (acc_ref[0, :N, :] + acc_ref[1, :N, :] + g2_ref[...]) * d
  out2 = out2 + b2_ref[...]
  m = jnp.sum(out2, axis=0, keepdims=True) * (1.0 / N)
  mask = lax.broadcasted_iota(jnp.int32, (1, 8), 1) < 6
  mx = jnp.max(jnp.where(mask, m, -1e30), axis=1, keepdims=True)
  z = m - mx
  s = jnp.sum(jnp.where(mask, jnp.exp(z), 0.0), axis=1, keepdims=True)
  out_ref[...] = z - jnp.log(s)


def kernel(x, edge_index, W1, b1, W2, b2):
  src = edge_index[0]
  dst = edge_index[1]
  pad = EP - E
  srcp = jnp.concatenate([src, jnp.zeros((pad,), jnp.int32)])
  dstp = jnp.concatenate([dst, jnp.full((pad,), N, jnp.int32)])
  src2d = srcp.reshape(CHUNKS, CHN)
  dst2d = dstp.reshape(CHUNKS, CHN)

  w2p = jnp.zeros((16, 8), jnp.float32).at[:, :6].set(W2)
  b2p = jnp.zeros((1, 8), jnp.float32).at[0, :6].set(b2)
  b1r = b1.reshape(1, 16)

  ones_deg = jnp.ones((CHN, DW), jnp.float32)
  zeros_deg = jnp.zeros((NP, DW), jnp.float32)
  zeros16 = jnp.zeros((NP, 16), jnp.float32)
  zeros8 = jnp.zeros((NP, 8), jnp.float32)

  deg_parts = _make_deg_kernel()(dst2d, ones_deg, zeros_deg)

  g1, d = pl.pallas_call(
      _prep_body,
      out_shape=[
          jax.ShapeDtypeStruct((N, 16), jnp.float32),
          jax.ShapeDtypeStruct((N, 1), jnp.float32),
      ],
  )(x, W1, deg_parts)

  acc1 = _make_agg_kernel(16)(src2d, dst2d, g1, zeros16)

  g2 = pl.pallas_call(
      _mid_body,
      out_shape=jax.ShapeDtypeStruct((N, 8), jnp.float32),
  )(acc1, g1, d, b1r, w2p)

  acc2 = _make_agg_kernel(8)(src2d, dst2d, g2, zeros8)

  outp = pl.pallas_call(
      _fin_body,
      out_shape=jax.ShapeDtypeStruct((1, 8), jnp.float32),
  )(acc2, g2, d, b2p)

  return outp[:, :6]


# speedup vs baseline: 1.0881x; 1.0129x over previous
import functools

import jax
import jax.numpy as jnp
from jax import lax
from jax.experimental import pallas as pl
from jax.experimental.pallas import tpu as pltpu
from jax.experimental.pallas import tpu_sc as plsc

NC = 2
NS = 16

N = 10000
NP = 10240
E = 320000
EP = 327680
CHN = 1024
CHUNKS = EP // CHN
RPS = NP // NS
DW = 8

SLOW_CID = 1
AGG_SLOW = 12
AGG_FAST = (CHUNKS - NS * AGG_SLOW) // NS
DEG_SLOW = 12
DEG_FAST = (CHUNKS - NS * DEG_SLOW) // NS
AGG_MAX = max(AGG_SLOW, AGG_FAST)
DEG_MAX = max(DEG_SLOW, DEG_FAST)


def _sc_mesh():
  return plsc.VectorSubcoreMesh(core_axis_name="c", subcore_axis_name="s")


def _zero_acc(acc_sh, zeros_hbm, sid):
  pltpu.sync_copy(zeros_hbm.at[pl.ds(sid * RPS, RPS)],
                  acc_sh.at[pl.ds(sid * RPS, RPS)])


def _copy_out(acc_sh, out_hbm, cid, sid):
  pltpu.sync_copy(acc_sh.at[pl.ds(sid * RPS, RPS)],
                  out_hbm.at[cid, pl.ds(sid * RPS, RPS)])


def _tile_span(cid, sid, n_slow, n_fast):
  slow = cid == SLOW_CID
  start = jnp.where(slow, sid * n_slow, NS * n_slow + sid * n_fast)
  count = jnp.where(slow, n_slow, n_fast)
  nmax = max(n_slow, n_fast)
  return start, jnp.minimum(start, CHUNKS - nmax), count


def _deg_body(dst_hbm, ones_hbm, zeros_hbm, out_hbm, dstv, ones_v, acc_sh,
              sem):
  cid = lax.axis_index("c")
  sid = lax.axis_index("s")
  _zero_acc(acc_sh, zeros_hbm, sid)
  start, cstart, count = _tile_span(cid, sid, DEG_SLOW, DEG_FAST)
  off = start - cstart
  pltpu.sync_copy(dst_hbm.at[pl.ds(cstart, DEG_MAX)], dstv)
  pltpu.sync_copy(ones_hbm, ones_v)
  plsc.subcore_barrier()

  def fire(c, _):
    pltpu.async_copy(ones_v, acc_sh.at[dstv.at[off + c]], sem, add=True)
    return 0

  def drain(c, _):
    pltpu.make_async_copy(ones_v, acc_sh.at[dstv.at[0]], sem).wait()
    return 0

  lax.fori_loop(0, count, fire, 0)
  lax.fori_loop(0, count, drain, 0)
  plsc.subcore_barrier()
  _copy_out(acc_sh, out_hbm, cid, sid)


def _make_deg_kernel():
  return pl.kernel(
      _deg_body,
      out_type=jax.ShapeDtypeStruct((NC, NP, DW), jnp.float32),
      mesh=_sc_mesh(),
      scratch_types=[
          pltpu.VMEM((DEG_MAX, CHN), jnp.int32),
          pltpu.VMEM((CHN, DW), jnp.float32),
          pltpu.VMEM_SHARED((NP, DW), jnp.float32),
          pltpu.SemaphoreType.DMA,
      ],
      compiler_params=pltpu.CompilerParams(use_tc_tiling_on_sc=False),
  )


def _agg_body(w, src_hbm, dst_hbm, g_hbm, zeros_hbm, out_hbm, srcv, dstv,
              rows, acc_sh, sg0, sg1, ss0, ss1):
  cid = lax.axis_index("c")
  sid = lax.axis_index("s")
  _zero_acc(acc_sh, zeros_hbm, sid)
  start, cstart, count = _tile_span(cid, sid, AGG_SLOW, AGG_FAST)
  off = start - cstart
  pltpu.sync_copy(src_hbm.at[pl.ds(cstart, AGG_MAX)], srcv)
  pltpu.sync_copy(dst_hbm.at[pl.ds(cstart, AGG_MAX)], dstv)
  plsc.subcore_barrier()

  def body(i, _):
    k0 = off + 2 * i
    k1 = k0 + 1

    @pl.when(i > 0)
    def _():
      pltpu.make_async_copy(rows.at[0], acc_sh.at[dstv.at[k0]], ss0).wait()

    pltpu.async_copy(g_hbm.at[srcv.at[k0]], rows.at[0], sg0)

    @pl.when(i > 0)
    def _():
      pltpu.make_async_copy(rows.at[1], acc_sh.at[dstv.at[k1]], ss1).wait()

    pltpu.async_copy(g_hbm.at[srcv.at[k1]], rows.at[1], sg1)

    pltpu.make_async_copy(g_hbm.at[srcv.at[k0]], rows.at[0], sg0).wait()
    pltpu.async_copy(rows.at[0], acc_sh.at[dstv.at[k0]], ss0, add=True)
    pltpu.make_async_copy(g_hbm.at[srcv.at[k1]], rows.at[1], sg1).wait()
    pltpu.async_copy(rows.at[1], acc_sh.at[dstv.at[k1]], ss1, add=True)
    return 0

  lax.fori_loop(0, count // 2, body, 0)
  pltpu.make_async_copy(rows.at[0], acc_sh.at[dstv.at[0]], ss0).wait()
  pltpu.make_async_copy(rows.at[1], acc_sh.at[dstv.at[1]], ss1).wait()
  plsc.subcore_barrier()
  _copy_out(acc_sh, out_hbm, cid, sid)


def _make_agg_kernel(w):
  return pl.kernel(
      functools.partial(_agg_body, w),
      out_type=jax.ShapeDtypeStruct((NC, NP, w), jnp.float32),
      mesh=_sc_mesh(),
      scratch_types=[
          pltpu.VMEM((AGG_MAX, CHN), jnp.int32),
          pltpu.VMEM((AGG_MAX, CHN), jnp.int32),
          pltpu.VMEM((2, CHN, w), jnp.float32),
          pltpu.VMEM_SHARED((NP, w), jnp.float32),
          pltpu.SemaphoreType.DMA,
          pltpu.SemaphoreType.DMA,
          pltpu.SemaphoreType.DMA,
          pltpu.SemaphoreType.DMA,
      ],
      compiler_params=pltpu.CompilerParams(use_tc_tiling_on_sc=False),
  )


def _prep_body(x_ref, w1_ref, degp_ref, g1_ref, d_ref):
  deg = degp_ref[0, :N, 0:1] + degp_ref[1, :N, 0:1] + 1.0
  d = lax.rsqrt(deg)
  h = jnp.dot(x_ref[...], w1_ref[...], preferred_element_type=jnp.float32)
  g1_ref[...] = h * d
  d_ref[...] = d


def _mid_body(acc_ref, g1_ref, d_ref, b1_ref, w2_ref, g2_ref):
  d = d_ref[...]
  acc = acc_ref[0, :N, :] + acc_ref[1, :N, :] + g1_ref[...]
  out1 = jnp.maximum(acc * d + b1_ref[...], 0.0)
  h2 = jnp.dot(out1, w2_ref[...], preferred_element_type=jnp.float32)
  g2_ref[...] = h2 * d


def _fin_body(acc_ref, g2_ref, d_ref, b2_ref, out_ref):
  d = d_ref[...]
  out2 = (acc_ref[0, :N, :] + acc_ref[1, :N, :] + g2_ref[...]) * d
  out2 = out2 + b2_ref[...]
  m = jnp.sum(out2, axis=0, keepdims=True) * (1.0 / N)
  mask = lax.broadcasted_iota(jnp.int32, (1, 8), 1) < 6
  mx = jnp.max(jnp.where(mask, m, -1e30), axis=1, keepdims=True)
  z = m - mx
  s = jnp.sum(jnp.where(mask, jnp.exp(z), 0.0), axis=1, keepdims=True)
  out_ref[...] = z - jnp.log(s)


def kernel(x, edge_index, W1, b1, W2, b2):
  src = edge_index[0]
  dst = edge_index[1]
  pad = EP - E
  srcp = jnp.concatenate([src, jnp.zeros((pad,), jnp.int32)])
  dstp = jnp.concatenate([dst, jnp.full((pad,), N, jnp.int32)])
  src2d = srcp.reshape(CHUNKS, CHN)
  dst2d = dstp.reshape(CHUNKS, CHN)

  w2p = jnp.zeros((16, 8), jnp.float32).at[:, :6].set(W2)
  b2p = jnp.zeros((1, 8), jnp.float32).at[0, :6].set(b2)
  b1r = b1.reshape(1, 16)

  ones_deg = jnp.ones((CHN, DW), jnp.float32)
  zeros_deg = jnp.zeros((NP, DW), jnp.float32)
  zeros16 = jnp.zeros((NP, 16), jnp.float32)
  zeros8 = jnp.zeros((NP, 8), jnp.float32)

  deg_parts = _make_deg_kernel()(dst2d, ones_deg, zeros_deg)

  g1, d = pl.pallas_call(
      _prep_body,
      out_shape=[
          jax.ShapeDtypeStruct((N, 16), jnp.float32),
          jax.ShapeDtypeStruct((N, 1), jnp.float32),
      ],
  )(x, W1, deg_parts)

  acc1 = _make_agg_kernel(16)(src2d, dst2d, g1, zeros16)

  g2 = pl.pallas_call(
      _mid_body,
      out_shape=jax.ShapeDtypeStruct((N, 8), jnp.float32),
  )(acc1, g1, d, b1r, w2p)

  acc2 = _make_agg_kernel(8)(src2d, dst2d, g2, zeros8)

  outp = pl.pallas_call(
      _fin_body,
      out_shape=jax.ShapeDtypeStruct((1, 8), jnp.float32),
  )(acc2, g2, d, b2p)

  return outp[:, :6]


# speedup vs baseline: 1.0978x; 1.0089x over previous
import functools

import jax
import jax.numpy as jnp
from jax import lax
from jax.experimental import pallas as pl
from jax.experimental.pallas import tpu as pltpu
from jax.experimental.pallas import tpu_sc as plsc

NC = 2
NS = 16

N = 10000
NP = 10240
E = 320000
EP = 327680
CHN = 1024
CHUNKS = EP // CHN
RPS = NP // NS
DW = 8

SLOW_CID = 1
AGG_SLOW = 14
AGG_FAST = (CHUNKS - NS * AGG_SLOW) // NS
DEG_SLOW = 14
DEG_FAST = (CHUNKS - NS * DEG_SLOW) // NS
AGG_MAX = max(AGG_SLOW, AGG_FAST)
DEG_MAX = max(DEG_SLOW, DEG_FAST)


def _sc_mesh():
  return plsc.VectorSubcoreMesh(core_axis_name="c", subcore_axis_name="s")


def _zero_acc(acc_sh, zeros_hbm, sid):
  pltpu.sync_copy(zeros_hbm.at[pl.ds(sid * RPS, RPS)],
                  acc_sh.at[pl.ds(sid * RPS, RPS)])


def _copy_out(acc_sh, out_hbm, cid, sid):
  pltpu.sync_copy(acc_sh.at[pl.ds(sid * RPS, RPS)],
                  out_hbm.at[cid, pl.ds(sid * RPS, RPS)])


def _tile_span(cid, sid, n_slow, n_fast):
  slow = cid == SLOW_CID
  start = jnp.where(slow, sid * n_slow, NS * n_slow + sid * n_fast)
  count = jnp.where(slow, n_slow, n_fast)
  nmax = max(n_slow, n_fast)
  return start, jnp.minimum(start, CHUNKS - nmax), count


def _deg_body(dst_hbm, ones_hbm, zeros_hbm, out_hbm, dstv, ones_v, acc_sh,
              sem):
  cid = lax.axis_index("c")
  sid = lax.axis_index("s")
  _zero_acc(acc_sh, zeros_hbm, sid)
  start, cstart, count = _tile_span(cid, sid, DEG_SLOW, DEG_FAST)
  off = start - cstart
  pltpu.sync_copy(dst_hbm.at[pl.ds(cstart, DEG_MAX)], dstv)
  pltpu.sync_copy(ones_hbm, ones_v)
  plsc.subcore_barrier()

  def fire(c, _):
    pltpu.async_copy(ones_v, acc_sh.at[dstv.at[off + c]], sem, add=True)
    return 0

  def drain(c, _):
    pltpu.make_async_copy(ones_v, acc_sh.at[dstv.at[0]], sem).wait()
    return 0

  lax.fori_loop(0, count, fire, 0)
  lax.fori_loop(0, count, drain, 0)
  plsc.subcore_barrier()
  _copy_out(acc_sh, out_hbm, cid, sid)


def _make_deg_kernel():
  return pl.kernel(
      _deg_body,
      out_type=jax.ShapeDtypeStruct((NC, NP, DW), jnp.float32),
      mesh=_sc_mesh(),
      scratch_types=[
          pltpu.VMEM((DEG_MAX, CHN), jnp.int32),
          pltpu.VMEM((CHN, DW), jnp.float32),
          pltpu.VMEM_SHARED((NP, DW), jnp.float32),
          pltpu.SemaphoreType.DMA,
      ],
      compiler_params=pltpu.CompilerParams(use_tc_tiling_on_sc=False),
  )


def _agg_body(w, src_hbm, dst_hbm, g_hbm, zeros_hbm, out_hbm, srcv, dstv,
              rows, acc_sh, sg0, sg1, ss0, ss1):
  cid = lax.axis_index("c")
  sid = lax.axis_index("s")
  _zero_acc(acc_sh, zeros_hbm, sid)
  start, cstart, count = _tile_span(cid, sid, AGG_SLOW, AGG_FAST)
  off = start - cstart
  pltpu.sync_copy(src_hbm.at[pl.ds(cstart, AGG_MAX)], srcv)
  pltpu.sync_copy(dst_hbm.at[pl.ds(cstart, AGG_MAX)], dstv)
  plsc.subcore_barrier()

  def body(i, _):
    k0 = off + 2 * i
    k1 = k0 + 1

    @pl.when(i > 0)
    def _():
      pltpu.make_async_copy(rows.at[0], acc_sh.at[dstv.at[k0]], ss0).wait()

    pltpu.async_copy(g_hbm.at[srcv.at[k0]], rows.at[0], sg0)

    @pl.when(i > 0)
    def _():
      pltpu.make_async_copy(rows.at[1], acc_sh.at[dstv.at[k1]], ss1).wait()

    pltpu.async_copy(g_hbm.at[srcv.at[k1]], rows.at[1], sg1)

    pltpu.make_async_copy(g_hbm.at[srcv.at[k0]], rows.at[0], sg0).wait()
    pltpu.async_copy(rows.at[0], acc_sh.at[dstv.at[k0]], ss0, add=True)
    pltpu.make_async_copy(g_hbm.at[srcv.at[k1]], rows.at[1], sg1).wait()
    pltpu.async_copy(rows.at[1], acc_sh.at[dstv.at[k1]], ss1, add=True)
    return 0

  lax.fori_loop(0, count // 2, body, 0)
  pltpu.make_async_copy(rows.at[0], acc_sh.at[dstv.at[0]], ss0).wait()
  pltpu.make_async_copy(rows.at[1], acc_sh.at[dstv.at[1]], ss1).wait()
  plsc.subcore_barrier()
  _copy_out(acc_sh, out_hbm, cid, sid)


def _make_agg_kernel(w):
  return pl.kernel(
      functools.partial(_agg_body, w),
      out_type=jax.ShapeDtypeStruct((NC, NP, w), jnp.float32),
      mesh=_sc_mesh(),
      scratch_types=[
          pltpu.VMEM((AGG_MAX, CHN), jnp.int32),
          pltpu.VMEM((AGG_MAX, CHN), jnp.int32),
          pltpu.VMEM((2, CHN, w), jnp.float32),
          pltpu.VMEM_SHARED((NP, w), jnp.float32),
          pltpu.SemaphoreType.DMA,
          pltpu.SemaphoreType.DMA,
          pltpu.SemaphoreType.DMA,
          pltpu.SemaphoreType.DMA,
      ],
      compiler_params=pltpu.CompilerParams(use_tc_tiling_on_sc=False),
  )


def _prep_body(x_ref, w1_ref, degp_ref, g1_ref, d_ref):
  deg = degp_ref[0, :N, 0:1] + degp_ref[1, :N, 0:1] + 1.0
  d = lax.rsqrt(deg)
  h = jnp.dot(x_ref[...], w1_ref[...], preferred_element_type=jnp.float32)
  g1_ref[...] = h * d
  d_ref[...] = d


def _mid_body(acc_ref, g1_ref, d_ref, b1_ref, w2_ref, g2_ref):
  d = d_ref[...]
  acc = acc_ref[0, :N, :] + acc_ref[1, :N, :] + g1_ref[...]
  out1 = jnp.maximum(acc * d + b1_ref[...], 0.0)
  h2 = jnp.dot(out1, w2_ref[...], preferred_element_type=jnp.float32)
  g2_ref[...] = h2 * d


def _fin_body(acc_ref, g2_ref, d_ref, b2_ref, out_ref):
  d = d_ref[...]
  out2 = (acc_ref[0, :N, :] + acc_ref[1, :N, :] + g2_ref[...]) * d
  out2 = out2 + b2_ref[...]
  m = jnp.sum(out2, axis=0, keepdims=True) * (1.0 / N)
  mask = lax.broadcasted_iota(jnp.int32, (1, 8), 1) < 6
  mx = jnp.max(jnp.where(mask, m, -1e30), axis=1, keepdims=True)
  z = m - mx
  s = jnp.sum(jnp.where(mask, jnp.exp(z), 0.0), axis=1, keepdims=True)
  out_ref[...] = z - jnp.log(s)


def kernel(x, edge_index, W1, b1, W2, b2):
  src = edge_index[0]
  dst = edge_index[1]
  pad = EP - E
  srcp = jnp.concatenate([src, jnp.zeros((pad,), jnp.int32)])
  dstp = jnp.concatenate([dst, jnp.full((pad,), N, jnp.int32)])
  src2d = srcp.reshape(CHUNKS, CHN)
  dst2d = dstp.reshape(CHUNKS, CHN)

  w2p = jnp.zeros((16, 8), jnp.float32).at[:, :6].set(W2)
  b2p = jnp.zeros((1, 8), jnp.float32).at[0, :6].set(b2)
  b1r = b1.reshape(1, 16)

  ones_deg = jnp.ones((CHN, DW), jnp.float32)
  zeros_deg = jnp.zeros((NP, DW), jnp.float32)
  zeros16 = jnp.zeros((NP, 16), jnp.float32)
  zeros8 = jnp.zeros((NP, 8), jnp.float32)

  deg_parts = _make_deg_kernel()(dst2d, ones_deg, zeros_deg)

  g1, d = pl.pallas_call(
      _prep_body,
      out_shape=[
          jax.ShapeDtypeStruct((N, 16), jnp.float32),
          jax.ShapeDtypeStruct((N, 1), jnp.float32),
      ],
  )(x, W1, deg_parts)

  acc1 = _make_agg_kernel(16)(src2d, dst2d, g1, zeros16)

  g2 = pl.pallas_call(
      _mid_body,
      out_shape=jax.ShapeDtypeStruct((N, 8), jnp.float32),
  )(acc1, g1, d, b1r, w2p)

  acc2 = _make_agg_kernel(8)(src2d, dst2d, g2, zeros8)

  outp = pl.pallas_call(
      _fin_body,
      out_shape=jax.ShapeDtypeStruct((1, 8), jnp.float32),
  )(acc2, g2, d, b2p)

  return outp[:, :6]
